# baseline (speedup 1.0000x reference)
import functools

import jax
import jax.numpy as jnp
from jax import lax
from jax.experimental import pallas as pl
from jax.experimental.pallas import tpu as pltpu
from jax.experimental.pallas import tpu_sc as plsc

DIM = 32
_INFO = plsc.get_sparse_core_info()
NC = _INFO.num_cores
NS = _INFO.num_subcores
NW = NC * NS
L16 = 16

CHUNK = 128


def _make_gather(n_total: int, vocab: int, nl: int):
  assert n_total == NW * CHUNK * nl
  bpw = n_total // NW
  mesh = plsc.VectorSubcoreMesh(core_axis_name="c", subcore_axis_name="s")

  @functools.partial(
      pl.kernel,
      mesh=mesh,
      compiler_params=pltpu.CompilerParams(use_tc_tiling_on_sc=False,
                                           needs_layout_passes=False),
      out_type=jax.ShapeDtypeStruct((nl, DIM, NW * CHUNK), jnp.float32),
      scratch_types=[
          pltpu.VMEM((bpw,), jnp.int32),
          pltpu.VMEM((CHUNK, DIM), jnp.float32),
          pltpu.VMEM((CHUNK, DIM), jnp.float32),
          pltpu.VMEM((DIM, CHUNK), jnp.float32),
          pltpu.VMEM((DIM, CHUNK), jnp.float32),
          pltpu.SemaphoreType.DMA,
          pltpu.SemaphoreType.DMA,
          pltpu.SemaphoreType.DMA,
          pltpu.SemaphoreType.DMA,
      ],
  )
  def gather_kernel(keys_hbm, table_hbm, out_hbm, idx_v, rows_a, rows_b,
                    tr_a, tr_b, sem_a, sem_b, osem_a, osem_b):
    wid = lax.axis_index("s") * NC + lax.axis_index("c")
    pltpu.sync_copy(keys_hbm.at[pl.ds(wid * bpw, bpw)], idx_v)
    bcol = wid * CHUNK

    def gather(c, buf, sem):
      return pltpu.async_copy(
          table_hbm.at[idx_v.at[pl.ds(c * CHUNK, CHUNK)]], buf, sem)

    def gather_wait(c, buf, sem):
      pltpu.make_async_copy(
          table_hbm.at[idx_v.at[pl.ds(c * CHUNK, CHUNK)]], buf, sem).wait()

    def transpose(src, dst):
      lanes = lax.iota(jnp.int32, L16)
      for g in range(CHUNK // L16):
        rows16 = lanes + (g * L16)
        for j in range(DIM):
          vals = plsc.load_gather(src, [rows16, jnp.full((L16,), j,
                                                         jnp.int32)])
          dst[j, pl.ds(g * L16, L16)] = vals

    def out_store(c, tr, osem):
      return pltpu.async_copy(
          tr, out_hbm.at[c, :, pl.ds(bcol, CHUNK)], osem)

    def out_wait(c, tr, osem):
      pltpu.make_async_copy(
          tr, out_hbm.at[c, :, pl.ds(bcol, CHUNK)], osem).wait()

    gather(0, rows_a, sem_a)
    gather(1, rows_b, sem_b)

    @pl.loop(0, nl, step=2)
    def _body(c):
      gather_wait(c, rows_a, sem_a)

      @pl.when(c >= 2)
      def _():
        out_wait(c - 2, tr_a, osem_a)

      transpose(rows_a, tr_a)
      out_store(c, tr_a, osem_a)

      @pl.when(c + 2 < nl)
      def _():
        gather(c + 2, rows_a, sem_a)

      gather_wait(c + 1, rows_b, sem_b)

      @pl.when(c >= 2)
      def _():
        out_wait(c - 1, tr_b, osem_b)

      transpose(rows_b, tr_b)
      out_store(c + 1, tr_b, osem_b)

      @pl.when(c + 3 < nl)
      def _():
        gather(c + 3, rows_b, sem_b)

    out_wait(nl - 2, tr_a, osem_a)
    out_wait(nl - 1, tr_b, osem_b)

  return gather_kernel


@jax.jit
def kernel(keys, table):
  b, l = keys.shape
  vocab, dim = table.shape
  n_total = b * l
  keys_m = keys.T.reshape(l, NW, CHUNK).transpose(1, 0, 2).reshape(-1)
  out = _make_gather(n_total, vocab, l)(keys_m, table)
  return jnp.transpose(out, (2, 0, 1))
